# Initial kernel scaffold; baseline (speedup 1.0000x reference)
import jax
import jax.numpy as jnp
from jax.experimental import pallas as pl


def kernel(feature, edge_index, W, b, gamma, beta):
    raise NotImplementedError("write your pallas kernel here")


# speedup vs baseline: 7.9166x; 7.9166x over previous
import functools

import jax
import jax.numpy as jnp
from jax import lax
from jax.experimental import pallas as pl
from jax.experimental.pallas import tpu as pltpu
from jax.experimental.pallas import tpu_sc as plsc

N = 10000
D = 128
E = 320000
EPS = 1e-5

N_PAD = 10240
CHUNK = 128
NUM_CORES = 2
NUM_TILES = 16
EDGES_PER_CORE = E // NUM_CORES
CHUNKS_PER_CORE = EDGES_PER_CORE // CHUNK
ROWS_PER_TILE = N_PAD // NUM_TILES

_mesh = plsc.VectorSubcoreMesh(core_axis_name="c", subcore_axis_name="s")


@functools.partial(
    pl.kernel,
    out_type=(
        jax.ShapeDtypeStruct((NUM_CORES, N_PAD, D), jnp.float32),
        jax.ShapeDtypeStruct((NUM_CORES, N_PAD), jnp.float32),
    ),
    mesh=_mesh,
    scratch_types=[
        pltpu.VMEM((CHUNK,), jnp.int32),
        pltpu.VMEM((CHUNK,), jnp.int32),
        pltpu.VMEM((CHUNK, D), jnp.float32),
        pltpu.VMEM((CHUNK,), jnp.float32),
        pltpu.VMEM((CHUNK,), jnp.float32),
        pltpu.VMEM_SHARED((N_PAD, D), jnp.float32),
        pltpu.VMEM_SHARED((N_PAD,), jnp.float32),
        pltpu.SemaphoreType.DMA,
    ],
)
def _sc_aggregate(feature_hbm, edge_hbm, sum_hbm, deg_hbm,
                  src_v, dst_v, rows_v, ones_v, zeros_v, acc_sh, deg_sh, sem):
    c = lax.axis_index("c")
    s = lax.axis_index("s")

    def zero_rows(r, carry):
        for k in range(D // 16):
            rows_v[r, pl.ds(k * 16, 16)] = jnp.zeros((16,), jnp.float32)
        return carry
    lax.fori_loop(0, CHUNK, zero_rows, 0)

    def init_small(r, carry):
        ones_v[pl.ds(r * 16, 16)] = jnp.ones((16,), jnp.float32)
        zeros_v[pl.ds(r * 16, 16)] = jnp.zeros((16,), jnp.float32)
        return carry
    lax.fori_loop(0, CHUNK // 16, init_small, 0)

    base = s * ROWS_PER_TILE
    for j in range(ROWS_PER_TILE // CHUNK):
        pltpu.sync_copy(rows_v, acc_sh.at[pl.ds(base + j * CHUNK, CHUNK)])
        pltpu.sync_copy(zeros_v, deg_sh.at[pl.ds(base + j * CHUNK, CHUNK)])
    plsc.subcore_barrier()

    n_chunks = 78 + jnp.where(s < CHUNKS_PER_CORE - 78 * NUM_TILES, 1, 0)

    def edge_body(i, carry):
        chunk = s + i * NUM_TILES
        start = c * EDGES_PER_CORE + chunk * CHUNK
        pltpu.sync_copy(edge_hbm.at[0, pl.ds(start, CHUNK)], src_v)
        pltpu.sync_copy(edge_hbm.at[1, pl.ds(start, CHUNK)], dst_v)
        pltpu.async_copy(feature_hbm.at[src_v], rows_v, sem).wait()
        pltpu.sync_copy(rows_v, acc_sh.at[dst_v], add=True)
        pltpu.sync_copy(ones_v, deg_sh.at[dst_v], add=True)
        return carry
    lax.fori_loop(0, n_chunks, edge_body, 0)
    plsc.subcore_barrier()

    pltpu.sync_copy(acc_sh.at[pl.ds(base, ROWS_PER_TILE)],
                    sum_hbm.at[c, pl.ds(base, ROWS_PER_TILE)])
    pltpu.sync_copy(deg_sh.at[pl.ds(base, ROWS_PER_TILE)],
                    deg_hbm.at[c, pl.ds(base, ROWS_PER_TILE)])


def _tc_body(psum_ref, pdeg_ref, feat_ref, w_ref, b_ref, gamma_ref, beta_ref,
             out_ref):
    ssum = psum_ref[0] + psum_ref[1]
    deg = pdeg_ref[0] + pdeg_ref[1]
    h = ssum[:N] / jnp.maximum(deg[:N], 1.0)
    h = lax.dot_general(h, w_ref[...], (((1,), (1,)), ((), ())),
                        preferred_element_type=jnp.float32)
    h = h + b_ref[...]
    mean = jnp.mean(h, axis=0, keepdims=True)
    var = jnp.mean((h - mean) ** 2, axis=0, keepdims=True)
    h = (h - mean) * (lax.rsqrt(var + EPS) * gamma_ref[...]) + beta_ref[...]
    out_ref[...] = feat_ref[...] + jnp.maximum(h, 0.0)


_tc_update = pl.pallas_call(
    _tc_body,
    out_shape=jax.ShapeDtypeStruct((N, D), jnp.float32),
)


def kernel(feature, edge_index, W, b, gamma, beta):
    psum, pdeg = _sc_aggregate(feature, edge_index)
    return _tc_update(psum, pdeg.reshape(NUM_CORES, N_PAD, 1), feature, W,
                      b.reshape(1, D), gamma.reshape(1, D), beta.reshape(1, D))
